# baseline (speedup 1.0000x reference)
import functools

import jax
import jax.numpy as jnp
from jax import lax
from jax.experimental import pallas as pl
from jax.experimental.pallas import tpu as pltpu
from jax.experimental.pallas import tpu_sc as plsc

_B, _T, _C = 4, 2048, 768
_K = 8
_FF = 4 * _C
_NKEY = 10
_NHALF = 5
_LK = 16
_BLK = 512


def _prestage_body(x_ref, rext_ref, rank_ref, val_ref):
    T, LK = _T, _LK
    x = x_ref[0]
    h = jnp.dot(x, rext_ref[...], preferred_element_type=jnp.float32)
    lane = lax.broadcasted_iota(jnp.int32, (T, LK), 1)
    hm = jnp.where(lane < _NKEY, h, jnp.float32(-1e30))
    rowmax = jnp.max(hm, axis=1, keepdims=True)
    idxf = jnp.min(jnp.where(hm >= rowmax, lane, LK), axis=1, keepdims=True)
    onehot = (lane == idxf).astype(jnp.float32)

    r128 = lax.broadcasted_iota(jnp.int32, (128, 128), 0)
    c128 = lax.broadcasted_iota(jnp.int32, (128, 128), 1)
    tril = (c128 <= r128).astype(jnp.bfloat16)
    pieces = []
    run = jnp.zeros((1, LK), jnp.float32)
    for p in range(T // 128):
        blk = onehot[p * 128:(p + 1) * 128, :].astype(jnp.bfloat16)
        cin = jnp.dot(tril, blk, preferred_element_type=jnp.float32)
        pieces.append(cin + run)
        run = run + cin[127:128, :]
    cum = jnp.concatenate(pieces, axis=0)
    counts = run
    cum_excl = cum - onehot
    below = jnp.sum(jnp.where(lane < idxf, jnp.broadcast_to(counts, (T, LK)), 0.0),
                    axis=1, keepdims=True)
    within = jnp.sum(onehot * cum_excl, axis=1, keepdims=True)
    rank = below + within
    rank_ref[0] = rank.astype(jnp.int32).reshape(1, T)
    val_ref[0] = x[:, 0:1].reshape(1, T)


def _prestage(x, rext):
    return pl.pallas_call(
        _prestage_body,
        grid=(_B,),
        in_specs=[
            pl.BlockSpec((1, _T, _C), lambda b: (b, 0, 0)),
            pl.BlockSpec((_C, _LK), lambda b: (0, 0)),
        ],
        out_specs=[
            pl.BlockSpec((1, 1, _T), lambda b: (b, 0, 0)),
            pl.BlockSpec((1, 1, _T), lambda b: (b, 0, 0)),
        ],
        out_shape=[
            jax.ShapeDtypeStruct((_B, 1, _T), jnp.int32),
            jax.ShapeDtypeStruct((_B, 1, _T), jnp.float32),
        ],
    )(x, rext)


def _sc_mix(rank, val, w16):
    mesh = plsc.VectorSubcoreMesh(core_axis_name="c", subcore_axis_name="s")

    @functools.partial(
        pl.kernel, mesh=mesh,
        out_type=jax.ShapeDtypeStruct((_B, _T), jnp.float32),
        compiler_params=pltpu.CompilerParams(needs_layout_passes=False),
        scratch_types=[
            pltpu.VMEM((_T,), jnp.int32),
            pltpu.VMEM((_T,), jnp.float32),
            pltpu.VMEM((_T + 16,), jnp.float32),
            pltpu.VMEM((_T,), jnp.float32),
            pltpu.VMEM((16,), jnp.float32),
            pltpu.SemaphoreType.DMA,
            pltpu.SemaphoreType.DMA,
            pltpu.SemaphoreType.DMA,
        ],
    )
    def k(rank_hbm, val_hbm, w_hbm, out_hbm, idx_v, v_v, g_v, hid_v, w_v,
          sem_r, sem_v, sem_w):
        wid = lax.axis_index("s") * 2 + lax.axis_index("c")

        @pl.when(wid < _B)
        def _():
            cr = pltpu.async_copy(rank_hbm.at[wid], idx_v, sem_r)
            cv = pltpu.async_copy(val_hbm.at[wid], v_v, sem_v)
            cw = pltpu.async_copy(w_hbm, w_v, sem_w)
            g_v[pl.ds(0, 16)] = jnp.zeros((16,), jnp.float32)
            cr.wait()
            cv.wait()
            cw.wait()

            @plsc.parallel_loop(0, _T // 16, unroll=8)
            def _(i):
                k16 = idx_v[pl.ds(i * 16, 16)] + 8
                v16 = v_v[pl.ds(i * 16, 16)]
                plsc.store_scatter(g_v, [k16], v16)

            wvec = w_v[...]
            lanes = lax.iota(jnp.int32, 16)
            ws = [jnp.sum(jnp.where(lanes == j, wvec, 0.0))
                  for j in range(_K + 1)]
            w0 = ws[0] + ws[_K]

            @plsc.parallel_loop(0, _T // 16, unroll=4)
            def _(i):
                base = i * 16 + 8
                acc = g_v[pl.ds(base, 16)] * w0
                for j in range(1, _K):
                    acc = acc + g_v[pl.ds(base - j, 16)] * ws[j]
                hid_v[pl.ds(i * 16, 16)] = acc

            pltpu.sync_copy(hid_v, out_hbm.at[wid])

    return k(rank, val, w16)


def _ffn_body(x_ref, hid_ref, gam_ref, bet_ref, w1_ref, b1_ref, w2_ref, b2_ref, o_ref):
    x = x_ref[...]
    out = x + hid_ref[...]
    mu = jnp.mean(out, axis=1, keepdims=True)
    d = out - mu
    var = jnp.mean(d * d, axis=1, keepdims=True)
    y = d * lax.rsqrt(var + 1e-5) * gam_ref[...] + bet_ref[...]
    h1 = lax.dot_general(y, w1_ref[...], (((1,), (1,)), ((), ())),
                         preferred_element_type=jnp.float32) + b1_ref[...]
    h1 = 0.5 * h1 * (1.0 + lax.erf(h1 * jnp.float32(0.7071067811865476)))
    h2 = lax.dot_general(h1, w2_ref[...], (((1,), (1,)), ((), ())),
                         preferred_element_type=jnp.float32) + b2_ref[...]
    o_ref[...] = h2 + out


def _ffn(x2, hid2, gam2, bet2, w1, b12, w2, b22):
    n = _B * _T
    return pl.pallas_call(
        _ffn_body,
        grid=(n // _BLK,),
        in_specs=[
            pl.BlockSpec((_BLK, _C), lambda i: (i, 0)),
            pl.BlockSpec((_BLK, 1), lambda i: (i, 0)),
            pl.BlockSpec((1, _C), lambda i: (0, 0)),
            pl.BlockSpec((1, _C), lambda i: (0, 0)),
            pl.BlockSpec((_FF, _C), lambda i: (0, 0)),
            pl.BlockSpec((1, _FF), lambda i: (0, 0)),
            pl.BlockSpec((_C, _FF), lambda i: (0, 0)),
            pl.BlockSpec((1, _C), lambda i: (0, 0)),
        ],
        out_specs=pl.BlockSpec((_BLK, _C), lambda i: (i, 0)),
        out_shape=jax.ShapeDtypeStruct((n, _C), jnp.float32),
    )(x2, hid2, gam2, bet2, w1, b12, w2, b22)


def kernel(input_tensor, random_R, kernel_total, ln_gamma, ln_beta, W1, b1, W2, b2):
    x = input_tensor
    rext = jnp.concatenate(
        [random_R, -random_R, jnp.zeros((_C, _LK - 2 * _NHALF), jnp.float32)], axis=1)
    w16 = jnp.pad(kernel_total.reshape(_K + 1), (0, _LK - (_K + 1)))
    rank, val = _prestage(x, rext)
    hid = _sc_mix(rank.reshape(_B, _T), val.reshape(_B, _T), w16)
    out = _ffn(x.reshape(_B * _T, _C), hid.reshape(_B * _T, 1),
               ln_gamma.reshape(1, _C), ln_beta.reshape(1, _C),
               W1, b1.reshape(1, _FF), W2, b2.reshape(1, _C))
    return out.reshape(_B, _T, _C)


# speedup vs baseline: 1.0043x; 1.0043x over previous
import functools

import jax
import jax.numpy as jnp
from jax import lax
from jax.experimental import pallas as pl
from jax.experimental.pallas import tpu as pltpu
from jax.experimental.pallas import tpu_sc as plsc

_B, _T, _C = 4, 2048, 768
_K = 8
_FF = 4 * _C
_NKEY = 10
_NHALF = 5
_LK = 16
_BLK = 1024


def _prestage_body(x_ref, rext_ref, rank_ref, val_ref):
    T, LK = _T, _LK
    x = x_ref[0]
    h = jnp.dot(x, rext_ref[...], preferred_element_type=jnp.float32)
    lane = lax.broadcasted_iota(jnp.int32, (T, LK), 1)
    hm = jnp.where(lane < _NKEY, h, jnp.float32(-1e30))
    rowmax = jnp.max(hm, axis=1, keepdims=True)
    idxf = jnp.min(jnp.where(hm >= rowmax, lane, LK), axis=1, keepdims=True)
    onehot = (lane == idxf).astype(jnp.float32)

    r128 = lax.broadcasted_iota(jnp.int32, (128, 128), 0)
    c128 = lax.broadcasted_iota(jnp.int32, (128, 128), 1)
    tril = (c128 <= r128).astype(jnp.bfloat16)
    pieces = []
    run = jnp.zeros((1, LK), jnp.float32)
    for p in range(T // 128):
        blk = onehot[p * 128:(p + 1) * 128, :].astype(jnp.bfloat16)
        cin = jnp.dot(tril, blk, preferred_element_type=jnp.float32)
        pieces.append(cin + run)
        run = run + cin[127:128, :]
    cum = jnp.concatenate(pieces, axis=0)
    counts = run
    cum_excl = cum - onehot
    below = jnp.sum(jnp.where(lane < idxf, jnp.broadcast_to(counts, (T, LK)), 0.0),
                    axis=1, keepdims=True)
    within = jnp.sum(onehot * cum_excl, axis=1, keepdims=True)
    rank = below + within
    rank_ref[0] = rank.astype(jnp.int32).reshape(1, T)
    val_ref[0] = x[:, 0:1].reshape(1, T)


def _prestage(x, rext):
    return pl.pallas_call(
        _prestage_body,
        grid=(_B,),
        in_specs=[
            pl.BlockSpec((1, _T, _C), lambda b: (b, 0, 0)),
            pl.BlockSpec((_C, _LK), lambda b: (0, 0)),
        ],
        out_specs=[
            pl.BlockSpec((1, 1, _T), lambda b: (b, 0, 0)),
            pl.BlockSpec((1, 1, _T), lambda b: (b, 0, 0)),
        ],
        out_shape=[
            jax.ShapeDtypeStruct((_B, 1, _T), jnp.int32),
            jax.ShapeDtypeStruct((_B, 1, _T), jnp.float32),
        ],
    )(x, rext)


def _sc_mix(rank, val, w16):
    mesh = plsc.VectorSubcoreMesh(core_axis_name="c", subcore_axis_name="s")

    @functools.partial(
        pl.kernel, mesh=mesh,
        out_type=jax.ShapeDtypeStruct((_B, _T), jnp.float32),
        compiler_params=pltpu.CompilerParams(needs_layout_passes=False),
        scratch_types=[
            pltpu.VMEM((_T,), jnp.int32),
            pltpu.VMEM((_T,), jnp.float32),
            pltpu.VMEM((_T + 16,), jnp.float32),
            pltpu.VMEM((_T,), jnp.float32),
            pltpu.VMEM((16,), jnp.float32),
            pltpu.SemaphoreType.DMA,
            pltpu.SemaphoreType.DMA,
            pltpu.SemaphoreType.DMA,
        ],
    )
    def k(rank_hbm, val_hbm, w_hbm, out_hbm, idx_v, v_v, g_v, hid_v, w_v,
          sem_r, sem_v, sem_w):
        wid = lax.axis_index("s") * 2 + lax.axis_index("c")

        @pl.when(wid < _B)
        def _():
            cr = pltpu.async_copy(rank_hbm.at[wid], idx_v, sem_r)
            cv = pltpu.async_copy(val_hbm.at[wid], v_v, sem_v)
            cw = pltpu.async_copy(w_hbm, w_v, sem_w)
            g_v[pl.ds(0, 16)] = jnp.zeros((16,), jnp.float32)
            cr.wait()
            cv.wait()
            cw.wait()

            @plsc.parallel_loop(0, _T // 16, unroll=8)
            def _(i):
                k16 = idx_v[pl.ds(i * 16, 16)] + 8
                v16 = v_v[pl.ds(i * 16, 16)]
                plsc.store_scatter(g_v, [k16], v16)

            wvec = w_v[...]
            lanes = lax.iota(jnp.int32, 16)
            ws = [jnp.sum(jnp.where(lanes == j, wvec, 0.0))
                  for j in range(_K + 1)]
            w0 = ws[0] + ws[_K]

            @plsc.parallel_loop(0, _T // 16, unroll=4)
            def _(i):
                base = i * 16 + 8
                acc = g_v[pl.ds(base, 16)] * w0
                for j in range(1, _K):
                    acc = acc + g_v[pl.ds(base - j, 16)] * ws[j]
                hid_v[pl.ds(i * 16, 16)] = acc

            pltpu.sync_copy(hid_v, out_hbm.at[wid])

    return k(rank, val, w16)


def _ffn_body(x_ref, hid_ref, gam_ref, bet_ref, w1_ref, b1_ref, w2_ref, b2_ref, o_ref):
    x = x_ref[...]
    out = x + hid_ref[...]
    mu = jnp.mean(out, axis=1, keepdims=True)
    d = out - mu
    var = jnp.mean(d * d, axis=1, keepdims=True)
    y = d * lax.rsqrt(var + 1e-5) * gam_ref[...] + bet_ref[...]
    h1 = lax.dot_general(y, w1_ref[...], (((1,), (1,)), ((), ())),
                         preferred_element_type=jnp.float32) + b1_ref[...]
    h1 = 0.5 * h1 * (1.0 + lax.erf(h1 * jnp.float32(0.7071067811865476)))
    h2 = lax.dot_general(h1, w2_ref[...], (((1,), (1,)), ((), ())),
                         preferred_element_type=jnp.float32) + b2_ref[...]
    o_ref[...] = h2 + out


def _ffn(x2, hid2, gam2, bet2, w1, b12, w2, b22):
    n = _B * _T
    return pl.pallas_call(
        _ffn_body,
        grid=(n // _BLK,),
        in_specs=[
            pl.BlockSpec((_BLK, _C), lambda i: (i, 0)),
            pl.BlockSpec((_BLK, 1), lambda i: (i, 0)),
            pl.BlockSpec((1, _C), lambda i: (0, 0)),
            pl.BlockSpec((1, _C), lambda i: (0, 0)),
            pl.BlockSpec((_FF, _C), lambda i: (0, 0)),
            pl.BlockSpec((1, _FF), lambda i: (0, 0)),
            pl.BlockSpec((_C, _FF), lambda i: (0, 0)),
            pl.BlockSpec((1, _C), lambda i: (0, 0)),
        ],
        out_specs=pl.BlockSpec((_BLK, _C), lambda i: (i, 0)),
        out_shape=jax.ShapeDtypeStruct((n, _C), jnp.float32),
    )(x2, hid2, gam2, bet2, w1, b12, w2, b22)


def kernel(input_tensor, random_R, kernel_total, ln_gamma, ln_beta, W1, b1, W2, b2):
    x = input_tensor
    rext = jnp.concatenate(
        [random_R, -random_R, jnp.zeros((_C, _LK - 2 * _NHALF), jnp.float32)], axis=1)
    w16 = jnp.pad(kernel_total.reshape(_K + 1), (0, _LK - (_K + 1)))
    rank, val = _prestage(x, rext)
    hid = _sc_mix(rank.reshape(_B, _T), val.reshape(_B, _T), w16)
    out = _ffn(x.reshape(_B * _T, _C), hid.reshape(_B * _T, 1),
               ln_gamma.reshape(1, _C), ln_beta.reshape(1, _C),
               W1, b1.reshape(1, _FF), W2, b2.reshape(1, _C))
    return out.reshape(_B, _T, _C)
